# baseline (speedup 1.0000x reference)
import functools

import jax
import jax.numpy as jnp
from jax import lax
from jax.experimental import pallas as pl
from jax.experimental.pallas import tpu as pltpu
from jax.experimental.pallas import tpu_sc as plsc

NC = 2
NS = 16
NW = NC * NS
LANES = 16
NBUF = 4
TC = 128


def _widen_table(table, tail_pad):
    V, D = table.shape
    tt = table.T
    n_chunks = V // TC
    tail = V - n_chunks * TC
    k_max = -(-n_chunks // NW)
    mesh = plsc.VectorSubcoreMesh(core_axis_name="c", subcore_axis_name="s")

    @functools.partial(
        pl.kernel,
        out_type=jax.ShapeDtypeStruct((V, 128), jnp.float32),
        mesh=mesh,
        compiler_params=pltpu.CompilerParams(
            use_tc_tiling_on_sc=True, needs_layout_passes=False),
        scratch_types=[
            pltpu.VMEM((2, D, TC), jnp.float32),
            pltpu.VMEM((2, TC, 128), jnp.float32),
            pltpu.VMEM((tail, 128), jnp.float32),
            pltpu.SemaphoreType.DMA,
            pltpu.SemaphoreType.DMA,
            pltpu.SemaphoreType.DMA,
            pltpu.SemaphoreType.DMA,
        ],
    )
    def widen(tt_hbm, tailp_hbm, tw_hbm, panel_v, out_v, tail_v,
              psem0, psem1, osem0, osem1):
        psems = (psem0, psem1)
        osems = (osem0, osem1)
        wid = lax.axis_index("s") * NC + lax.axis_index("c")

        def cid_of(k):
            return k * NW + wid

        def start_panel(k, b):
            for fr in range(D // 8):
                pltpu.async_copy(
                    tt_hbm.at[pl.ds(fr * 8, 8), pl.ds(cid_of(k) * TC, TC)],
                    panel_v.at[b, pl.ds(fr * 8, 8)], psems[b])

        iotas = [lax.iota(jnp.int32, LANES) + c * LANES
                 for c in range(D // LANES)]

        @pl.when(cid_of(0) < n_chunks)
        def _():
            start_panel(0, 0)

        def body(g, carry):
          for b in range(2):
            k = g * 2 + b
            valid = cid_of(k) < n_chunks

            @pl.when(valid)
            def _():
                @pl.when(k >= 2)
                def _():
                    pltpu.make_async_copy(
                        out_v.at[b], tw_hbm.at[pl.ds(0, TC)],
                        osems[b]).wait()

                pltpu.make_async_copy(
                    tt_hbm.at[:, pl.ds(0, TC)], panel_v.at[b],
                    psems[b]).wait()

                @pl.when(cid_of(k + 1) < n_chunks)
                def _():
                    start_panel(k + 1, 1 - b)

                def rows(u, carry2):
                    base = jnp.full((LANES,), u * 16, jnp.int32)
                    for j in range(16):
                        r = u * 16 + j
                        colv = base + j
                        for c in range(D // LANES):
                            vals = plsc.load_gather(
                                panel_v.at[b], [iotas[c], colv])
                            out_v[b, r, pl.ds(c * LANES, LANES)] = vals
                    return carry2

                lax.fori_loop(0, TC // 16, rows, None)
                pltpu.async_copy(
                    out_v.at[b], tw_hbm.at[pl.ds(cid_of(k) * TC, TC)],
                    osems[b])
          return carry

        lax.fori_loop(0, -(-k_max // 2), body, None)
        for b in range(2):
            pltpu.make_async_copy(
                out_v.at[b], tw_hbm.at[pl.ds(0, TC)], osems[b]).wait()

        @pl.when(wid == 0)
        def _():
            pltpu.sync_copy(tailp_hbm, tail_v)
            pltpu.sync_copy(tail_v, tw_hbm.at[pl.ds(n_chunks * TC, tail)])

    return widen(tt, tail_pad)


def kernel(inputs, table, bias):
    B, L = inputs.shape
    V, D = table.shape
    PAIR = 2
    GW = PAIR * L
    GWP = GW + (-GW % 8)
    n_groups = B // PAIR
    g_per_w = n_groups // NW
    bags_per_w = B // NW
    n_col = D // LANES

    tail = V % TC
    tail_pad = jnp.pad(table[V - tail:], ((0, 0), (0, 128 - D)))
    table_w = _widen_table(table, tail_pad)

    idx = inputs.astype(jnp.int32).reshape(n_groups, GW)
    npad = GWP - GW
    pad = (jnp.arange(n_groups, dtype=jnp.int32)[:, None] * npad
           + jnp.arange(npad, dtype=jnp.int32)[None, :]) % V
    idx = jnp.concatenate([idx, pad], axis=1).reshape(-1)

    mesh = plsc.VectorSubcoreMesh(core_axis_name="c", subcore_axis_name="s")

    @functools.partial(
        pl.kernel,
        out_type=jax.ShapeDtypeStruct((B * D,), jnp.float32),
        mesh=mesh,
        compiler_params=pltpu.CompilerParams(use_tc_tiling_on_sc=True),
        scratch_types=[
            pltpu.VMEM((g_per_w * GWP,), jnp.int32),
            pltpu.VMEM((NBUF, GWP, 128), jnp.float32),
            pltpu.VMEM((bags_per_w * D,), jnp.float32),
            pltpu.VMEM((D,), jnp.float32),
        ] + [pltpu.SemaphoreType.DMA] * NBUF,
    )
    def bow(table_hbm, idx_hbm, bias_hbm, out_hbm,
            idx_v, rows_v, out_v, bias_v, *sems):
        wid = lax.axis_index("s") * NC + lax.axis_index("c")
        gbase = wid * g_per_w

        pltpu.sync_copy(bias_hbm, bias_v)
        pltpu.sync_copy(idx_hbm.at[pl.ds(gbase * GWP, g_per_w * GWP)], idx_v)

        def start(s, b):
            pltpu.async_copy(
                table_hbm.at[idx_v.at[pl.ds(s * GWP, GWP)]], rows_v.at[b],
                sems[b])

        def wait(b):
            pltpu.make_async_copy(
                table_hbm.at[idx_v.at[pl.ds(0, GWP)]], rows_v.at[b],
                sems[b]).wait()

        def reduce_step(s, b):
            for p in range(PAIR):
                for c in range(n_col):
                    acc = bias_v[pl.ds(c * LANES, LANES)]
                    for l in range(L):
                        acc = acc + rows_v[b, p * L + l, pl.ds(c * LANES, LANES)]
                    out_v[pl.ds((s * PAIR + p) * D + c * LANES, LANES)] = acc

        for b in range(NBUF):
            start(b, b)

        def body(g, carry):
            for b in range(NBUF):
                s = g * NBUF + b
                wait(b)
                reduce_step(s, b)
                s2 = s + NBUF

                @pl.when(s2 < g_per_w)
                def _():
                    start(s2, b)
            return carry

        lax.fori_loop(0, g_per_w // NBUF, body, None)
        pltpu.sync_copy(
            out_v, out_hbm.at[pl.ds(wid * bags_per_w * D, bags_per_w * D)])

    return bow(table_w, idx, bias).reshape(B, D)


# speedup vs baseline: 2.0857x; 2.0857x over previous
import functools

import jax
import jax.numpy as jnp
from jax import lax
from jax.experimental import pallas as pl
from jax.experimental.pallas import tpu as pltpu
from jax.experimental.pallas import tpu_sc as plsc

NC = 2
NS = 16
NW = NC * NS
LANES = 16
NBUF = 4


def kernel(inputs, table, bias):
    B, L = inputs.shape
    V, D = table.shape
    PAIR = 2
    GW = PAIR * L
    GWP = GW + (-GW % 8)
    n_groups = B // PAIR
    g_per_w = n_groups // NW
    bags_per_w = B // NW
    n_col = D // LANES

    table_w = jnp.pad(table, ((0, 0), (0, 128 - D)))

    idx = inputs.astype(jnp.int32).reshape(n_groups, GW)
    npad = GWP - GW
    pad = (jnp.arange(n_groups, dtype=jnp.int32)[:, None] * npad
           + jnp.arange(npad, dtype=jnp.int32)[None, :]) % V
    idx = jnp.concatenate([idx, pad], axis=1).reshape(-1)

    mesh = plsc.VectorSubcoreMesh(core_axis_name="c", subcore_axis_name="s")

    @functools.partial(
        pl.kernel,
        out_type=jax.ShapeDtypeStruct((B * D,), jnp.float32),
        mesh=mesh,
        compiler_params=pltpu.CompilerParams(use_tc_tiling_on_sc=True),
        scratch_types=[
            pltpu.VMEM((g_per_w * GWP,), jnp.int32),
            pltpu.VMEM((NBUF, GWP, 128), jnp.float32),
            pltpu.VMEM((bags_per_w * D,), jnp.float32),
            pltpu.VMEM((D,), jnp.float32),
        ] + [pltpu.SemaphoreType.DMA] * NBUF,
    )
    def bow(table_hbm, idx_hbm, bias_hbm, out_hbm,
            idx_v, rows_v, out_v, bias_v, *sems):
        wid = lax.axis_index("s") * NC + lax.axis_index("c")
        gbase = wid * g_per_w

        pltpu.sync_copy(bias_hbm, bias_v)
        pltpu.sync_copy(idx_hbm.at[pl.ds(gbase * GWP, g_per_w * GWP)], idx_v)

        def start(s, b):
            pltpu.async_copy(
                table_hbm.at[idx_v.at[pl.ds(s * GWP, GWP)]], rows_v.at[b],
                sems[b])

        def wait(b):
            pltpu.make_async_copy(
                table_hbm.at[idx_v.at[pl.ds(0, GWP)]], rows_v.at[b],
                sems[b]).wait()

        def reduce_step(s, b):
            for p in range(PAIR):
                for c in range(n_col):
                    acc = bias_v[pl.ds(c * LANES, LANES)]
                    for l in range(L):
                        acc = acc + rows_v[b, p * L + l, pl.ds(c * LANES, LANES)]
                    out_v[pl.ds((s * PAIR + p) * D + c * LANES, LANES)] = acc

        for b in range(NBUF):
            start(b, b)

        def body(g, carry):
            for b in range(NBUF):
                s = g * NBUF + b
                wait(b)
                reduce_step(s, b)
                s2 = s + NBUF

                @pl.when(s2 < g_per_w)
                def _():
                    start(s2, b)
            return carry

        lax.fori_loop(0, g_per_w // NBUF, body, None)
        pltpu.sync_copy(
            out_v, out_hbm.at[pl.ds(wid * bags_per_w * D, bags_per_w * D)])

    return bow(table_w, idx, bias).reshape(B, D)
